# baseline (speedup 1.0000x reference)
import functools
import math

import numpy as np
import jax
import jax.numpy as jnp
from jax.experimental import pallas as pl
from jax.experimental.pallas import tpu as pltpu

LEAKY_SLOPE = 0.1


def _leaky(x):
    return jnp.where(x >= 0, x, LEAKY_SLOPE * x)


def _interp_matrix_np(out_size, in_size):
    if in_size == 1:
        return np.ones((out_size, 1), np.float32)
    denom = max(out_size - 1, 1)
    idx = np.arange(out_size, dtype=np.float64)
    src = idx * (in_size - 1) / denom
    lo = np.clip(np.floor(src).astype(np.int64), 0, in_size - 2)
    frac = (src - lo).astype(np.float32)
    A = np.zeros((out_size, in_size), np.float32)
    rows = np.arange(out_size)
    np.add.at(A, (rows, lo), 1.0 - frac)
    np.add.at(A, (rows, lo + 1), frac)
    return A


def _kron_padded_np(H, W, h, w, Wp, Pz):
    Ah = _interp_matrix_np(H, h)
    Aw = _interp_matrix_np(W, w)
    K = np.kron(Ah, Aw).T.astype(np.float32)
    Kp = np.zeros((h * w, Pz), np.float32)
    for y in range(H):
        Kp[:, (y + 1) * Wp + 1:(y + 1) * Wp + 1 + W] = K[:, y * W:(y + 1) * W]
    return Kp


def _fused_kernel(xpf_ref, xh_ref, w1_0t_ref, w1_1t_ref,
                  k2_ref, k4_ref, w3c_ref, wft_ref, o_ref, z_ref, zc_ref,
                  *, wp, pv):
    C = xpf_ref.shape[1]
    Cb = w1_0t_ref.shape[0]
    H = xh_ref.shape[1]
    W = xh_ref.shape[2]
    z_ref[0:C, :] = xpf_ref[0]
    xh = xh_ref[0].reshape(H // 2, 2, W // 2, 2, C)
    p2 = jnp.max(xh, axis=(1, 3))
    p4r = p2.reshape(H // 4, 2, W // 4, 2, C)
    p4 = jnp.max(p4r, axis=(1, 3)).reshape((H // 4) * (W // 4), C)
    p2 = p2.reshape((H // 2) * (W // 2), C)
    dn = (((1,), (1,)), ((), ()))
    ys2 = jax.lax.dot_general(w1_0t_ref[...], p2, dn,
                              preferred_element_type=jnp.float32)
    up2 = jnp.dot(ys2.astype(jnp.bfloat16), k2_ref[...],
                  preferred_element_type=jnp.float32)
    z_ref[C:C + Cb, :] = _leaky(up2).astype(jnp.bfloat16)
    ys4 = jax.lax.dot_general(w1_1t_ref[...], p4, dn,
                              preferred_element_type=jnp.float32)
    up4 = jnp.dot(ys4.astype(jnp.bfloat16), k4_ref[...],
                  preferred_element_type=jnp.float32)
    z_ref[C + Cb:C + 2 * Cb, :] = _leaky(up4).astype(jnp.bfloat16)
    Ct = z_ref.shape[0]
    for ty in range(3):
        for tx in range(3):
            t = ty * 3 + tx
            off = ty * wp + tx
            zc_ref[t * Ct:(t + 1) * Ct, :] = z_ref[:, off:off + pv]
    acc = jnp.dot(w3c_ref[...], zc_ref[...], preferred_element_type=jnp.float32)
    acc = _leaky(acc).astype(jnp.bfloat16)
    y = jnp.dot(wft_ref[...], acc, preferred_element_type=jnp.float32)
    o_ref[0] = y


@jax.jit
def kernel(x, w1_0, w1_1, w3, wf):
    N, C, H, W = x.shape
    Cb = w1_0.shape[1]
    Ct = C + 2 * Cb
    Cout = w3.shape[2]
    Wp = W + 2
    Pv = H * Wp
    Pz = (H + 2) * Wp + 2

    xb = x.astype(jnp.bfloat16)
    xp = jnp.pad(xb, ((0, 0), (0, 0), (1, 1), (1, 1))).reshape(N, C, (H + 2) * Wp)
    xpf = jnp.pad(xp, ((0, 0), (0, 0), (0, 2)))
    xh = jnp.transpose(xb, (0, 2, 3, 1))

    k2 = jnp.asarray(_kron_padded_np(H, W, H // 2, W // 2, Wp, Pz), jnp.bfloat16)
    k4 = jnp.asarray(_kron_padded_np(H, W, H // 4, W // 4, Wp, Pz), jnp.bfloat16)

    w1_0t = w1_0.T.astype(jnp.bfloat16)
    w1_1t = w1_1.T.astype(jnp.bfloat16)
    w3c = jnp.transpose(w3, (2, 0, 1)).reshape(Cout, 9 * Ct).astype(jnp.bfloat16)
    wft = wf.T.astype(jnp.bfloat16)

    kern = functools.partial(_fused_kernel, wp=Wp, pv=Pv)
    out_flat = pl.pallas_call(
        kern,
        out_shape=jax.ShapeDtypeStruct((N, Cout, Pv), jnp.float32),
        grid=(2, N // 2),
        in_specs=[
            pl.BlockSpec((1, C, Pz), lambda i, j: (i * (N // 2) + j, 0, 0)),
            pl.BlockSpec((1, H, W, C), lambda i, j: (i * (N // 2) + j, 0, 0, 0)),
            pl.BlockSpec((Cb, C), lambda i, j: (0, 0)),
            pl.BlockSpec((Cb, C), lambda i, j: (0, 0)),
            pl.BlockSpec(((H // 2) * (W // 2), Pz), lambda i, j: (0, 0)),
            pl.BlockSpec(((H // 4) * (W // 4), Pz), lambda i, j: (0, 0)),
            pl.BlockSpec((Cout, 9 * Ct), lambda i, j: (0, 0)),
            pl.BlockSpec((Cout, Cout), lambda i, j: (0, 0)),
        ],
        out_specs=pl.BlockSpec((1, Cout, Pv), lambda i, j: (i * (N // 2) + j, 0, 0)),
        scratch_shapes=[pltpu.VMEM((Ct, Pz), jnp.bfloat16),
                        pltpu.VMEM((9 * Ct, Pv), jnp.bfloat16)],
        compiler_params=pltpu.CompilerParams(
            dimension_semantics=("parallel", "arbitrary"),
            vmem_limit_bytes=56 * 1024 * 1024),
    )(xpf, xh, w1_0t, w1_1t, k2, k4, w3c, wft)

    return out_flat.reshape(N, Cout, H, Wp)[:, :, :, :W]


# speedup vs baseline: 1.0946x; 1.0946x over previous
import functools
import math

import numpy as np
import jax
import jax.numpy as jnp
from jax.experimental import pallas as pl
from jax.experimental.pallas import tpu as pltpu

LEAKY_SLOPE = 0.1


def _leaky(x):
    return jnp.where(x >= 0, x, LEAKY_SLOPE * x)


def _interp_matrix_np(out_size, in_size):
    if in_size == 1:
        return np.ones((out_size, 1), np.float32)
    denom = max(out_size - 1, 1)
    idx = np.arange(out_size, dtype=np.float64)
    src = idx * (in_size - 1) / denom
    lo = np.clip(np.floor(src).astype(np.int64), 0, in_size - 2)
    frac = (src - lo).astype(np.float32)
    A = np.zeros((out_size, in_size), np.float32)
    rows = np.arange(out_size)
    np.add.at(A, (rows, lo), 1.0 - frac)
    np.add.at(A, (rows, lo + 1), frac)
    return A


def _kron_padded_np(H, W, h, w, Wp, Pz):
    Ah = _interp_matrix_np(H, h)
    Aw = _interp_matrix_np(W, w)
    K = np.kron(Ah, Aw).T.astype(np.float32)
    Kp = np.zeros((h * w, Pz), np.float32)
    for y in range(H):
        Kp[:, (y + 1) * Wp + 1:(y + 1) * Wp + 1 + W] = K[:, y * W:(y + 1) * W]
    return Kp


def _fused_kernel(xpf_ref, w1_0t_ref, w1_1t_ref, s2_ref, s4_ref,
                  k2_ref, k4_ref, w3t_ref, wft_ref, o_ref, z_ref,
                  *, wp, pv, lm, lm4):
    C = xpf_ref.shape[1]
    Cb = w1_0t_ref.shape[0]
    z_ref[0:C, :] = xpf_ref[0]
    zx = xpf_ref[0]
    m = jnp.maximum(jnp.maximum(zx[:, 0:lm], zx[:, 1:lm + 1]),
                    jnp.maximum(zx[:, wp:lm + wp], zx[:, wp + 1:lm + wp + 1]))
    m44 = jnp.maximum(
        jnp.maximum(m[:, 0:lm4], m[:, 2:lm4 + 2]),
        jnp.maximum(m[:, 2 * wp:lm4 + 2 * wp], m[:, 2 * wp + 2:lm4 + 2 * wp + 2]))
    q2 = jnp.dot(w1_0t_ref[...], m, preferred_element_type=jnp.float32)
    ys2 = jnp.dot(q2.astype(jnp.bfloat16), s2_ref[...],
                  preferred_element_type=jnp.float32)
    up2 = jnp.dot(ys2.astype(jnp.bfloat16), k2_ref[...],
                  preferred_element_type=jnp.float32)
    z_ref[C:C + Cb, :] = _leaky(up2).astype(jnp.bfloat16)
    q4 = jnp.dot(w1_1t_ref[...], m44, preferred_element_type=jnp.float32)
    ys4 = jnp.dot(q4.astype(jnp.bfloat16), s4_ref[...],
                  preferred_element_type=jnp.float32)
    up4 = jnp.dot(ys4.astype(jnp.bfloat16), k4_ref[...],
                  preferred_element_type=jnp.float32)
    z_ref[C + Cb:C + 2 * Cb, :] = _leaky(up4).astype(jnp.bfloat16)
    acc = None
    for ty in range(3):
        for tx in range(3):
            t = ty * 3 + tx
            off = ty * wp + tx
            tap = z_ref[:, off:off + pv]
            d = jnp.dot(w3t_ref[t], tap, preferred_element_type=jnp.float32)
            acc = d if acc is None else acc + d
    acc = _leaky(acc).astype(jnp.bfloat16)
    y = jnp.dot(wft_ref[...], acc, preferred_element_type=jnp.float32)
    o_ref[0] = y


@jax.jit
def kernel(x, w1_0, w1_1, w3, wf):
    N, C, H, W = x.shape
    Cb = w1_0.shape[1]
    Ct = C + 2 * Cb
    Cout = w3.shape[2]
    Wp = W + 2
    Pv = H * Wp
    Pz = (H + 2) * Wp + 2

    xb = x.astype(jnp.bfloat16)
    xp = jnp.pad(xb, ((0, 0), (0, 0), (1, 1), (1, 1))).reshape(N, C, (H + 2) * Wp)
    xpf = jnp.pad(xp, ((0, 0), (0, 0), (0, 2)))

    Lm = Pz - Wp - 1
    Lm4 = Lm - 2 * Wp - 2
    h2, w2 = H // 2, W // 2
    h4, w4 = H // 4, W // 4
    S2 = np.zeros((Lm, h2 * w2), np.float32)
    for i in range(h2):
        for j in range(w2):
            S2[Wp * (2 * i + 1) + 2 * j + 1, i * w2 + j] = 1.0
    S4 = np.zeros((Lm4, h4 * w4), np.float32)
    for i in range(h4):
        for j in range(w4):
            S4[Wp * (4 * i + 1) + 4 * j + 1, i * w4 + j] = 1.0
    s2 = jnp.asarray(S2, jnp.bfloat16)
    s4 = jnp.asarray(S4, jnp.bfloat16)

    k2 = jnp.asarray(_kron_padded_np(H, W, H // 2, W // 2, Wp, Pz), jnp.bfloat16)
    k4 = jnp.asarray(_kron_padded_np(H, W, H // 4, W // 4, Wp, Pz), jnp.bfloat16)

    w1_0t = w1_0.T.astype(jnp.bfloat16)
    w1_1t = w1_1.T.astype(jnp.bfloat16)
    w3t = jnp.transpose(w3, (0, 2, 1)).astype(jnp.bfloat16)
    wft = wf.T.astype(jnp.bfloat16)

    kern = functools.partial(_fused_kernel, wp=Wp, pv=Pv, lm=Lm, lm4=Lm4)
    out_flat = pl.pallas_call(
        kern,
        out_shape=jax.ShapeDtypeStruct((N, Cout, Pv), jnp.float32),
        grid=(N,),
        in_specs=[
            pl.BlockSpec((1, C, Pz), lambda n: (n, 0, 0)),
            pl.BlockSpec((Cb, C), lambda n: (0, 0)),
            pl.BlockSpec((Cb, C), lambda n: (0, 0)),
            pl.BlockSpec((Lm, (H // 2) * (W // 2)), lambda n: (0, 0)),
            pl.BlockSpec((Lm4, (H // 4) * (W // 4)), lambda n: (0, 0)),
            pl.BlockSpec(((H // 2) * (W // 2), Pz), lambda n: (0, 0)),
            pl.BlockSpec(((H // 4) * (W // 4), Pz), lambda n: (0, 0)),
            pl.BlockSpec((9, Cout, Ct), lambda n: (0, 0, 0)),
            pl.BlockSpec((Cout, Cout), lambda n: (0, 0)),
        ],
        out_specs=pl.BlockSpec((1, Cout, Pv), lambda n: (n, 0, 0)),
        scratch_shapes=[pltpu.VMEM((Ct, Pz), jnp.bfloat16)],
        compiler_params=pltpu.CompilerParams(
            dimension_semantics=("parallel",),
            vmem_limit_bytes=56 * 1024 * 1024),
    )(xpf, w1_0t, w1_1t, s2, s4, k2, k4, w3t, wft)

    return out_flat.reshape(N, Cout, H, Wp)[:, :, :, :W]
